# baseline (speedup 1.0000x reference)
import functools

import jax
import jax.numpy as jnp
from jax import lax
from jax.experimental import pallas as pl
from jax.experimental.pallas import tpu as pltpu
from jax.experimental.pallas import tpu_sc as plsc

N = 10000
E = 320000
D = 128
DH = D // 2

NC = 2
NS = 16

CHUNK = 128
NBUF = 2
NCH = 160
NCHH = NCH // 2
EPT = NCH * CHUNK
E_PAD = NS * EPT
SCH = NCH + NBUF
SPT = SCH * CHUNK
N_PAD = 10112
ROWS_PER_TILE = N_PAD // NS
NVEC = SPT // 16

_mesh = plsc.VectorSubcoreMesh(core_axis_name="c", subcore_axis_name="s")
_sc_params = pltpu.CompilerParams(use_tc_tiling_on_sc=False)


@functools.partial(
    pl.kernel,
    out_type=jax.ShapeDtypeStruct((NC * N_PAD, 16), jnp.float32),
    mesh=_mesh,
    scratch_types=[
        pltpu.VMEM((NCH, CHUNK), jnp.int32),
        pltpu.VMEM((CHUNK, 16), jnp.float32),
        pltpu.VMEM((CHUNK, 16), jnp.float32),
        pltpu.VMEM_SHARED((N_PAD, 16), jnp.float32),
    ],
    compiler_params=_sc_params,
)
def _sc_degree(dst_hbm, out, dst_v, ones_v, zeros_v, deg_sh):
    c = lax.axis_index("c")
    s = lax.axis_index("s")

    one16 = jnp.ones((16,), jnp.float32)
    zero16 = jnp.zeros((16,), jnp.float32)

    def fill(i, _):
        ones_v[i, :] = one16
        zeros_v[i, :] = zero16
        return 0

    lax.fori_loop(0, CHUNK, fill, 0)

    base = s * ROWS_PER_TILE
    nfull = ROWS_PER_TILE // CHUNK
    for k in range(nfull):
        pltpu.sync_copy(zeros_v, deg_sh.at[pl.ds(base + k * CHUNK, CHUNK)])
    rem = ROWS_PER_TILE - nfull * CHUNK
    if rem:
        pltpu.sync_copy(zeros_v.at[pl.ds(0, rem)],
                        deg_sh.at[pl.ds(base + nfull * CHUNK, rem)])

    pltpu.sync_copy(dst_hbm.at[s], dst_v)
    plsc.subcore_barrier()

    lo = c * NCHH

    def body(j, _):
        pltpu.sync_copy(ones_v, deg_sh.at[dst_v.at[lo + j]], add=True)
        return 0

    lax.fori_loop(0, NCHH, body, 0)
    plsc.subcore_barrier()

    pltpu.sync_copy(deg_sh.at[pl.ds(base, ROWS_PER_TILE)],
                    out.at[pl.ds(c * N_PAD + base, ROWS_PER_TILE)])


@functools.partial(
    pl.kernel,
    out_type=jax.ShapeDtypeStruct((NC * N_PAD, DH), jnp.float32),
    mesh=_mesh,
    scratch_types=[
        pltpu.VMEM((SPT,), jnp.int32),
        pltpu.VMEM((NCH, CHUNK), jnp.int32),
        pltpu.VMEM((CHUNK, DH), jnp.float32),
        pltpu.VMEM((CHUNK, DH), jnp.float32),
        pltpu.VMEM_SHARED((N_PAD, DH), jnp.float32),
        pltpu.SemaphoreType.DMA,
    ],
    compiler_params=_sc_params,
)
def _sc_scatter(g_hbm, src_hbm, dst_hbm, out,
                src_v, dst_v, gbuf, zeros_v, acc_sh, sem):
    c = lax.axis_index("c")
    s = lax.axis_index("s")

    zero16 = jnp.zeros((16,), jnp.float32)

    def fill(i, _):
        for j in range(DH // 16):
            zeros_v[i, pl.ds(j * 16, 16)] = zero16
        return 0

    lax.fori_loop(0, CHUNK, fill, 0)

    base = s * ROWS_PER_TILE
    nfull = ROWS_PER_TILE // CHUNK
    for k in range(nfull):
        pltpu.sync_copy(zeros_v, acc_sh.at[pl.ds(base + k * CHUNK, CHUNK)])
    rem = ROWS_PER_TILE - nfull * CHUNK
    if rem:
        pltpu.sync_copy(zeros_v.at[pl.ds(0, rem)],
                        acc_sh.at[pl.ds(base + nfull * CHUNK, rem)])

    pltpu.sync_copy(src_hbm.at[c * NS + s], src_v)
    pltpu.sync_copy(dst_hbm.at[s], dst_v)
    plsc.subcore_barrier()

    def body(j, _):
        pltpu.async_copy(g_hbm.at[src_v.at[pl.ds(j * CHUNK, CHUNK)]],
                         gbuf, sem).wait()
        pltpu.sync_copy(gbuf, acc_sh.at[dst_v.at[j]], add=True)
        return 0

    lax.fori_loop(0, NCH, body, 0)
    plsc.subcore_barrier()

    pltpu.sync_copy(acc_sh.at[pl.ds(base, ROWS_PER_TILE)],
                    out.at[pl.ds(c * N_PAD + base, ROWS_PER_TILE)])


_RB = 1264
_GRID = N_PAD // _RB


def _dinv(deg_ref):
    deg = deg_ref[0, :, 0:1] + deg_ref[1, :, 0:1] + 1.0
    return lax.rsqrt(deg)


def _cat(ref):
    a = ref[...]
    return jnp.concatenate([a[0], a[1]], axis=-1)


def _store_split(ref, val):
    ref[0] = val[:, :DH]
    ref[1] = val[:, DH:]


def _tc1_body(x_ref, w1_ref, deg_ref, g1_ref):
    dinv = _dinv(deg_ref)
    res = dinv * jnp.dot(x_ref[...], w1_ref[...],
                         preferred_element_type=jnp.float32)
    _store_split(g1_ref, res)


def _tc2_body(acc_ref, g1_ref, deg_ref, b1_ref, w2_ref, g2_ref):
    dinv = _dinv(deg_ref)
    z = dinv * (_cat(acc_ref) + _cat(g1_ref)) + b1_ref[...]
    r = jnp.maximum(z, 0.0)
    res = dinv * jnp.dot(r, w2_ref[...], preferred_element_type=jnp.float32)
    _store_split(g2_ref, res)


def _tc3_body(acc_ref, g2_ref, deg_ref, b2_ref, out_ref):
    dinv = _dinv(deg_ref)
    out_ref[...] = dinv * (_cat(acc_ref) + _cat(g2_ref)) + b2_ref[...]


_row_spec = pl.BlockSpec((_RB, D), lambda i: (i, 0))
_half_spec = pl.BlockSpec((NC, _RB, DH), lambda i: (0, i, 0))
_deg_spec = pl.BlockSpec((NC, _RB, 16), lambda i: (0, i, 0))
_w_spec = pl.BlockSpec((D, D), lambda i: (0, 0))
_b_spec = pl.BlockSpec((1, D), lambda i: (0, 0))
_f32 = jnp.float32

_split_shape = jax.ShapeDtypeStruct((NC, N_PAD, DH), _f32)

_tc1 = pl.pallas_call(
    _tc1_body,
    grid=(_GRID,),
    in_specs=[_row_spec, _w_spec, _deg_spec],
    out_specs=_half_spec,
    out_shape=_split_shape,
)

_tc2 = pl.pallas_call(
    _tc2_body,
    grid=(_GRID,),
    in_specs=[_half_spec, _half_spec, _deg_spec, _b_spec, _w_spec],
    out_specs=_half_spec,
    out_shape=_split_shape,
)

_tc3 = pl.pallas_call(
    _tc3_body,
    grid=(_GRID,),
    in_specs=[_half_spec, _half_spec, _deg_spec, _b_spec],
    out_specs=_row_spec,
    out_shape=jax.ShapeDtypeStruct((N_PAD, D), _f32),
)


def kernel(x, edge_index, W1, b1, W2, b2):
    src = edge_index[0].astype(jnp.int32)
    dst = edge_index[1].astype(jnp.int32)
    src_p = jnp.pad(
        jnp.zeros((E_PAD,), jnp.int32).at[:E].set(src).reshape(NS, EPT),
        ((0, 0), (0, SPT - EPT)))
    src_p = jnp.stack([src_p, src_p + N_PAD]).reshape(NC * NS, SPT)
    dst_p = jnp.full((E_PAD,), N, jnp.int32).at[:E].set(dst).reshape(
        NS, NCH, CHUNK)
    x_pad = jnp.pad(x, ((0, N_PAD - N), (0, 0)))
    b1r = b1.reshape(1, D)
    b2r = b2.reshape(1, D)

    deg = _sc_degree(dst_p).reshape(NC, N_PAD, 16)
    g1 = _tc1(x_pad, W1, deg)
    acc1 = _sc_scatter(g1.reshape(NC * N_PAD, DH), src_p, dst_p)
    g2 = _tc2(acc1.reshape(NC, N_PAD, DH), g1, deg, b1r, W2)
    acc2 = _sc_scatter(g2.reshape(NC * N_PAD, DH), src_p, dst_p)
    out = _tc3(acc2.reshape(NC, N_PAD, DH), g2, deg, b2r)
    return out[:N]


# speedup vs baseline: 1.3760x; 1.3760x over previous
import functools

import jax
import jax.numpy as jnp
from jax import lax
from jax.experimental import pallas as pl
from jax.experimental.pallas import tpu as pltpu
from jax.experimental.pallas import tpu_sc as plsc

N = 10000
E = 320000
D = 128
DH = D // 2

NC = 2
NS = 16

CHUNK = 128
NCH = 158
NCHH = NCH // 2
EPT = NCH * CHUNK
E_PAD = NS * EPT
N_PAD = 10112
ROWS_PER_TILE = N_PAD // NS
NVEC = EPT // 16

_mesh = plsc.VectorSubcoreMesh(core_axis_name="c", subcore_axis_name="s")
_sc_params = pltpu.CompilerParams(use_tc_tiling_on_sc=False)


@functools.partial(
    pl.kernel,
    out_type=jax.ShapeDtypeStruct((NC * N_PAD, 16), jnp.float32),
    mesh=_mesh,
    scratch_types=[
        pltpu.VMEM((NCH, CHUNK), jnp.int32),
        pltpu.VMEM((CHUNK, 16), jnp.float32),
        pltpu.VMEM((CHUNK, 16), jnp.float32),
        pltpu.VMEM_SHARED((N_PAD, 16), jnp.float32),
    ],
    compiler_params=_sc_params,
)
def _sc_degree(dst_hbm, out, dst_v, ones_v, zeros_v, deg_sh):
    c = lax.axis_index("c")
    s = lax.axis_index("s")

    one16 = jnp.ones((16,), jnp.float32)
    zero16 = jnp.zeros((16,), jnp.float32)

    def fill(i, _):
        ones_v[i, :] = one16
        zeros_v[i, :] = zero16
        return 0

    lax.fori_loop(0, CHUNK, fill, 0)

    base = s * ROWS_PER_TILE
    nfull = ROWS_PER_TILE // CHUNK
    for k in range(nfull):
        pltpu.sync_copy(zeros_v, deg_sh.at[pl.ds(base + k * CHUNK, CHUNK)])
    rem = ROWS_PER_TILE - nfull * CHUNK
    if rem:
        pltpu.sync_copy(zeros_v.at[pl.ds(0, rem)],
                        deg_sh.at[pl.ds(base + nfull * CHUNK, rem)])

    pltpu.sync_copy(dst_hbm.at[s], dst_v)
    plsc.subcore_barrier()

    lo = c * NCHH

    def body(j, _):
        pltpu.sync_copy(ones_v, deg_sh.at[dst_v.at[lo + j]], add=True)
        return 0

    lax.fori_loop(0, NCHH, body, 0)
    plsc.subcore_barrier()

    pltpu.sync_copy(deg_sh.at[pl.ds(base, ROWS_PER_TILE)],
                    out.at[pl.ds(c * N_PAD + base, ROWS_PER_TILE)])


@functools.partial(
    pl.kernel,
    out_type=jax.ShapeDtypeStruct((NC * N_PAD, DH), jnp.float32),
    mesh=_mesh,
    scratch_types=[
        pltpu.VMEM((EPT,), jnp.int32),
        pltpu.VMEM((NCH, CHUNK), jnp.int32),
        pltpu.VMEM((CHUNK, DH), jnp.float32),
        pltpu.VMEM((CHUNK, DH), jnp.float32),
        pltpu.VMEM_SHARED((N_PAD, DH), jnp.float32),
        pltpu.SemaphoreType.DMA,
    ],
    compiler_params=_sc_params,
)
def _sc_scatter(g_hbm, src_hbm, dst_hbm, out,
                src_v, dst_v, rows_v, zeros_v, acc_sh, sem):
    c = lax.axis_index("c")
    s = lax.axis_index("s")

    zero16 = jnp.zeros((16,), jnp.float32)

    def fill(i, _):
        for j in range(DH // 16):
            zeros_v[i, pl.ds(j * 16, 16)] = zero16
        return 0

    lax.fori_loop(0, CHUNK, fill, 0)

    base = s * ROWS_PER_TILE
    nfull = ROWS_PER_TILE // CHUNK
    for k in range(nfull):
        pltpu.sync_copy(zeros_v, acc_sh.at[pl.ds(base + k * CHUNK, CHUNK)])
    rem = ROWS_PER_TILE - nfull * CHUNK
    if rem:
        pltpu.sync_copy(zeros_v.at[pl.ds(0, rem)],
                        acc_sh.at[pl.ds(base + nfull * CHUNK, rem)])

    pltpu.sync_copy(src_hbm.at[s], src_v)
    pltpu.sync_copy(dst_hbm.at[s], dst_v)

    off = c * N_PAD

    def adjust(i, _):
        src_v[pl.ds(i * 16, 16)] = src_v[pl.ds(i * 16, 16)] + off
        return 0

    lax.fori_loop(0, NVEC, adjust, 0)
    plsc.subcore_barrier()

    def body(j, _):
        pltpu.async_copy(g_hbm.at[src_v.at[pl.ds(j * CHUNK, CHUNK)]],
                         rows_v, sem).wait()
        pltpu.sync_copy(rows_v, acc_sh.at[dst_v.at[j]], add=True)
        return 0

    lax.fori_loop(0, NCH, body, 0)
    plsc.subcore_barrier()

    pltpu.sync_copy(acc_sh.at[pl.ds(base, ROWS_PER_TILE)],
                    out.at[pl.ds(c * N_PAD + base, ROWS_PER_TILE)])


_RB = 1264
_GRID = N_PAD // _RB


def _dinv(deg_ref):
    deg = deg_ref[0, :, 0:1] + deg_ref[1, :, 0:1] + 1.0
    return lax.rsqrt(deg)


def _cat(ref):
    a = ref[...]
    return jnp.concatenate([a[0], a[1]], axis=-1)


def _store_split(ref, val):
    ref[0] = val[:, :DH]
    ref[1] = val[:, DH:]


def _tc1_body(x_ref, w1_ref, deg_ref, g1_ref):
    dinv = _dinv(deg_ref)
    res = dinv * jnp.dot(x_ref[...], w1_ref[...],
                         preferred_element_type=jnp.float32)
    _store_split(g1_ref, res)


def _tc2_body(acc_ref, g1_ref, deg_ref, b1_ref, w2_ref, g2_ref):
    dinv = _dinv(deg_ref)
    z = dinv * (_cat(acc_ref) + _cat(g1_ref)) + b1_ref[...]
    r = jnp.maximum(z, 0.0)
    res = dinv * jnp.dot(r, w2_ref[...], preferred_element_type=jnp.float32)
    _store_split(g2_ref, res)


def _tc3_body(acc_ref, g2_ref, deg_ref, b2_ref, out_ref):
    dinv = _dinv(deg_ref)
    out_ref[...] = dinv * (_cat(acc_ref) + _cat(g2_ref)) + b2_ref[...]


_row_spec = pl.BlockSpec((_RB, D), lambda i: (i, 0))
_half_spec = pl.BlockSpec((NC, _RB, DH), lambda i: (0, i, 0))
_deg_spec = pl.BlockSpec((NC, _RB, 16), lambda i: (0, i, 0))
_w_spec = pl.BlockSpec((D, D), lambda i: (0, 0))
_b_spec = pl.BlockSpec((1, D), lambda i: (0, 0))
_f32 = jnp.float32

_split_shape = jax.ShapeDtypeStruct((NC, N_PAD, DH), _f32)

_tc1 = pl.pallas_call(
    _tc1_body,
    grid=(_GRID,),
    in_specs=[_row_spec, _w_spec, _deg_spec],
    out_specs=_half_spec,
    out_shape=_split_shape,
)

_tc2 = pl.pallas_call(
    _tc2_body,
    grid=(_GRID,),
    in_specs=[_half_spec, _half_spec, _deg_spec, _b_spec, _w_spec],
    out_specs=_half_spec,
    out_shape=_split_shape,
)

_tc3 = pl.pallas_call(
    _tc3_body,
    grid=(_GRID,),
    in_specs=[_half_spec, _half_spec, _deg_spec, _b_spec],
    out_specs=_row_spec,
    out_shape=jax.ShapeDtypeStruct((N_PAD, D), _f32),
)


def kernel(x, edge_index, W1, b1, W2, b2):
    src = edge_index[0].astype(jnp.int32)
    dst = edge_index[1].astype(jnp.int32)
    src_p = jnp.zeros((E_PAD,), jnp.int32).at[:E].set(src).reshape(NS, EPT)
    dst_p = jnp.full((E_PAD,), N, jnp.int32).at[:E].set(dst).reshape(
        NS, NCH, CHUNK)
    x_pad = jnp.pad(x, ((0, N_PAD - N), (0, 0)))
    b1r = b1.reshape(1, D)
    b2r = b2.reshape(1, D)

    deg = _sc_degree(dst_p).reshape(NC, N_PAD, 16)
    g1 = _tc1(x_pad, W1, deg)
    acc1 = _sc_scatter(g1.reshape(NC * N_PAD, DH), src_p, dst_p)
    g2 = _tc2(acc1.reshape(NC, N_PAD, DH), g1, deg, b1r, W2)
    acc2 = _sc_scatter(g2.reshape(NC * N_PAD, DH), src_p, dst_p)
    out = _tc3(acc2.reshape(NC, N_PAD, DH), g2, deg, b2r)
    return out[:N]
